# baseline (speedup 1.0000x reference)
import jax
import jax.numpy as jnp
from jax.experimental import pallas as pl
from jax.experimental.pallas import tpu as pltpu

_LSPLIT = 2


def _concat_body(pos_ref, h_ref, emb_ref, out_ref):
    s = h_ref.shape[1]
    out_ref[0, pl.ds(1, s), :] = h_ref[0]
    out_ref[0, pl.ds(0, 1), :] = emb_ref[pl.ds(pos_ref[0], 1), :]


def kernel(hidden_states, position, embeddings):
    b, s, d = hidden_states.shape
    depth = embeddings.shape[0]
    dl = d // _LSPLIT
    pos_arr = jnp.asarray(position, jnp.int32).reshape((1,))
    return pl.pallas_call(
        _concat_body,
        grid=(b, _LSPLIT),
        out_shape=jax.ShapeDtypeStruct((b, s + 1, d), hidden_states.dtype),
        in_specs=[
            pl.BlockSpec(memory_space=pltpu.SMEM),
            pl.BlockSpec((1, s, dl), lambda i, j: (i, 0, j)),
            pl.BlockSpec((depth, dl), lambda i, j: (0, j)),
        ],
        out_specs=pl.BlockSpec((1, s + 1, dl), lambda i, j: (i, 0, j)),
    )(pos_arr, hidden_states, embeddings)
